# baseline (speedup 1.0000x reference)
import functools
import jax
import jax.numpy as jnp
from jax import lax
from jax.experimental import pallas as pl
from jax.experimental.pallas import tpu as pltpu
from jax.experimental.pallas import tpu_sc as plsc

BS = 128
VOCAB = 100000
CH = 20000
NCHUNK = VOCAB // CH
LANES = 16

NEG_BIG = -3.0e38
IDX_BIG = 2 ** 30

_info = plsc.get_sparse_core_info()
NC = _info.num_cores
NS = _info.num_subcores
NW = NC * NS
ROWS_PER_W = BS // NW

_mesh = plsc.VectorSubcoreMesh(core_axis_name="c", subcore_axis_name="s")


@functools.partial(
    pl.kernel,
    mesh=_mesh,
    out_type=(
        jax.ShapeDtypeStruct((BS * LANES,), jnp.float32),
        jax.ShapeDtypeStruct((BS * LANES,), jnp.int32),
    ),
    scratch_types=[
        pltpu.VMEM((CH,), jnp.float32),
        pltpu.VMEM((CH,), jnp.int32),
        pltpu.VMEM((LANES,), jnp.float32),
        pltpu.VMEM((LANES,), jnp.float32),
        pltpu.VMEM((LANES,), jnp.int32),
    ],
)
def _sc_scan(l_hbm, t_hbm, pen_hbm, val_hbm, idx_hbm, lbuf, tbuf, penv, vbuf, obuf):
    wid = lax.axis_index("s") * NC + lax.axis_index("c")
    lane = lax.iota(jnp.int32, LANES)
    for r_local in range(ROWS_PER_W):
        row = wid * ROWS_PER_W + r_local
        pltpu.sync_copy(pen_hbm.at[pl.ds(row * LANES, LANES)], penv)
        pen_s = penv[...]
        rp_s = 1.0 / pen_s
        bv = jnp.full((LANES,), NEG_BIG, jnp.float32)
        bi = jnp.full((LANES,), 0, jnp.int32)
        for c in range(NCHUNK):
            pltpu.sync_copy(l_hbm.at[pl.ds(row * VOCAB + c * CH, CH)], lbuf)
            pltpu.sync_copy(t_hbm.at[pl.ds(row * VOCAB + c * CH, CH)], tbuf)
            base = c * CH

            def body(j, carry, base=base):
                v, i = carry
                l = lbuf[pl.ds(j * LANES, LANES)]
                t = tbuf[pl.ds(j * LANES, LANES)]
                pm = jnp.minimum(l * rp_s, l * pen_s)
                p = jnp.where(t > 0, pm, l)
                idx = base + j * LANES + lane
                better = p > v
                return (jnp.where(better, p, v), jnp.where(better, idx, i))

            bv, bi = lax.fori_loop(0, CH // LANES, body, (bv, bi))
        vbuf[...] = bv
        obuf[...] = bi
        pltpu.sync_copy(vbuf, val_hbm.at[pl.ds(row * LANES, LANES)])
        pltpu.sync_copy(obuf, idx_hbm.at[pl.ds(row * LANES, LANES)])


def _merge_body(v_ref, i_ref, o_ref):
    v = v_ref[...]
    i = i_ref[...]
    m = jnp.max(v, axis=1, keepdims=True)
    cand = jnp.where(v == m, i, IDX_BIG)
    o_ref[...] = jnp.min(cand, axis=1, keepdims=True)


def kernel(logits, repetition_penalty, token_count):
    l = logits.reshape(BS * VOCAB)
    t = token_count.reshape(BS * VOCAB)
    pen = jnp.broadcast_to(repetition_penalty.reshape(BS, 1), (BS, LANES)).reshape(BS * LANES)
    vals, idxs = _sc_scan(l, t, pen)
    out = pl.pallas_call(
        _merge_body,
        out_shape=jax.ShapeDtypeStruct((BS, 1), jnp.int32),
    )(vals.reshape(BS, LANES), idxs.reshape(BS, LANES))
    return out


# speedup vs baseline: 2.2959x; 2.2959x over previous
import functools
import jax
import jax.numpy as jnp
from jax import lax
from jax.experimental import pallas as pl
from jax.experimental.pallas import tpu as pltpu
from jax.experimental.pallas import tpu_sc as plsc

BS = 128
VOCAB = 100000
LANES = 16
STRIPE = 3200
NVEC = STRIPE // LANES
UNROLL = 4
NSTEP = NVEC // UNROLL
NCHUNK = BS // 8
TILES_TOTAL = 782
LAST_TILE_OFF = TILES_TOTAL - STRIPE // 128

NEG_BIG = -3.0e38
IDX_BIG = 2 ** 30

_info = plsc.get_sparse_core_info()
NC = _info.num_cores
NS = _info.num_subcores
NW = NC * NS

_mesh = plsc.VectorSubcoreMesh(core_axis_name="c", subcore_axis_name="s")


@functools.partial(
    pl.kernel,
    mesh=_mesh,
    out_type=(
        jax.ShapeDtypeStruct((BS * NW * LANES,), jnp.float32),
        jax.ShapeDtypeStruct((BS * NW * LANES,), jnp.int32),
    ),
    scratch_types=[
        pltpu.VMEM((2, 8, STRIPE), jnp.float32),
        pltpu.VMEM((2, 8, STRIPE), jnp.int32),
        pltpu.VMEM((BS * LANES,), jnp.float32),
        pltpu.VMEM((BS * LANES,), jnp.float32),
        pltpu.VMEM((BS * LANES,), jnp.int32),
        pltpu.SemaphoreType.DMA,
        pltpu.SemaphoreType.DMA,
        pltpu.SemaphoreType.DMA,
        pltpu.SemaphoreType.DMA,
        pltpu.SemaphoreType.DMA,
        pltpu.SemaphoreType.DMA,
    ],
)
def _sc_scan(l_hbm, t_hbm, pen_hbm, val_hbm, idx_hbm,
             lbuf, tbuf, penv, pv, pi,
             lsem0, lsem1, tsem0, tsem1, vsem, isem):
    wid = lax.axis_index("s") * NC + lax.axis_index("c")
    lane = lax.iota(jnp.int32, LANES)
    off = pl.multiple_of((wid * LAST_TILE_OFF) // (NW - 1) * 128, 128)
    lsems = (lsem0, lsem1)
    tsems = (tsem0, tsem1)

    pltpu.sync_copy(pen_hbm, penv)

    def lsrc(c):
        return l_hbm.at[pl.ds(pl.multiple_of(c * 8, 8), 8), pl.ds(off, STRIPE)]

    def tsrc(c):
        return t_hbm.at[pl.ds(pl.multiple_of(c * 8, 8), 8), pl.ds(off, STRIPE)]

    def start(c, b):
        pltpu.async_copy(lsrc(c), lbuf.at[b], lsems[b])
        pltpu.async_copy(tsrc(c), tbuf.at[b], tsems[b])

    def wait(c, b):
        pltpu.make_async_copy(lsrc(c), lbuf.at[b], lsems[b]).wait()
        pltpu.make_async_copy(tsrc(c), tbuf.at[b], tsems[b]).wait()

    start(0, 0)

    def chunk_body(g, carry):
        for b in range(2):
            c = g * 2 + b
            wait(c, b)

            @pl.when(c + 1 < NCHUNK)
            def _():
                start(c + 1, 1 - b)

            for r8 in range(8):
                row = c * 8 + r8
                pen_s = penv[pl.ds(row * LANES, LANES)]
                rp_s = 1.0 / pen_s

                def body(j, acc, b=b, r8=r8):
                    new = []
                    for u in range(UNROLL):
                        v, i = acc[2 * u], acc[2 * u + 1]
                        col = j * (UNROLL * LANES) + u * LANES
                        l = lbuf[b, r8, pl.ds(col, LANES)]
                        t = tbuf[b, r8, pl.ds(col, LANES)]
                        pm = jnp.minimum(l * rp_s, l * pen_s)
                        p = jnp.where(t > 0, pm, l)
                        gidx = off + col + lane
                        better = jnp.logical_and(p > v, gidx < VOCAB)
                        new.append(jnp.where(better, p, v))
                        new.append(jnp.where(better, gidx, i))
                    return tuple(new)

                acc0 = []
                for _u in range(UNROLL):
                    acc0.append(jnp.full((LANES,), NEG_BIG, jnp.float32))
                    acc0.append(jnp.full((LANES,), 0, jnp.int32))
                acc = lax.fori_loop(0, NSTEP, body, tuple(acc0))

                bv, bi = acc[0], acc[1]
                for u in range(1, UNROLL):
                    v2, i2 = acc[2 * u], acc[2 * u + 1]
                    take = jnp.logical_or(
                        v2 > bv, jnp.logical_and(v2 == bv, i2 < bi))
                    bv = jnp.where(take, v2, bv)
                    bi = jnp.where(take, i2, bi)

                pv[pl.ds(row * LANES, LANES)] = bv
                pi[pl.ds(row * LANES, LANES)] = bi
                dst = row * (NW * LANES) + wid * LANES
                pltpu.async_copy(pv.at[pl.ds(row * LANES, LANES)],
                                 val_hbm.at[pl.ds(dst, LANES)], vsem)
                pltpu.async_copy(pi.at[pl.ds(row * LANES, LANES)],
                                 idx_hbm.at[pl.ds(dst, LANES)], isem)
        return carry

    lax.fori_loop(0, NCHUNK // 2, chunk_body, 0)

    pltpu.make_async_copy(pv, val_hbm.at[pl.ds(0, BS * LANES)], vsem).wait()
    pltpu.make_async_copy(pi, idx_hbm.at[pl.ds(0, BS * LANES)], isem).wait()


def _merge_body(v_ref, i_ref, o_ref):
    v = v_ref[...]
    i = i_ref[...]
    m = jnp.max(v, axis=1, keepdims=True)
    cand = jnp.where(v == m, i, IDX_BIG)
    o_ref[...] = jnp.min(cand, axis=1, keepdims=True)


def kernel(logits, repetition_penalty, token_count):
    l = logits.reshape(BS, VOCAB)
    t = token_count.reshape(BS, VOCAB)
    pen = jnp.broadcast_to(repetition_penalty.reshape(BS, 1),
                           (BS, LANES)).reshape(BS * LANES)
    vals, idxs = _sc_scan(l, t, pen)
    out = pl.pallas_call(
        _merge_body,
        out_shape=jax.ShapeDtypeStruct((BS, 1), jnp.int32),
    )(vals.reshape(BS, NW * LANES), idxs.reshape(BS, NW * LANES))
    return out


# speedup vs baseline: 2.3747x; 1.0343x over previous
import functools
import jax
import jax.numpy as jnp
from jax import lax
from jax.experimental import pallas as pl
from jax.experimental.pallas import tpu as pltpu
from jax.experimental.pallas import tpu_sc as plsc

BS = 128
VOCAB = 100000
LANES = 16
STRIPE = 3200
NVEC = STRIPE // LANES
UNROLL = 4
NSTEP = NVEC // UNROLL
NCHUNK = BS // 8
TILES_TOTAL = 782
LAST_TILE_OFF = TILES_TOTAL - STRIPE // 128

NEG_BIG = -3.0e38
IDX_BIG = 2 ** 30

_info = plsc.get_sparse_core_info()
NC = _info.num_cores
NS = _info.num_subcores
NW = NC * NS

_mesh = plsc.VectorSubcoreMesh(core_axis_name="c", subcore_axis_name="s")


@functools.partial(
    pl.kernel,
    mesh=_mesh,
    out_type=(
        jax.ShapeDtypeStruct((NW * BS,), jnp.float32),
        jax.ShapeDtypeStruct((NW * BS,), jnp.int32),
    ),
    scratch_types=[
        pltpu.VMEM((2, 8, STRIPE), jnp.float32),
        pltpu.VMEM((2, 8, STRIPE), jnp.int32),
        pltpu.VMEM((BS * LANES,), jnp.float32),
        pltpu.VMEM((BS,), jnp.float32),
        pltpu.VMEM((BS,), jnp.int32),
        pltpu.SemaphoreType.DMA,
        pltpu.SemaphoreType.DMA,
        pltpu.SemaphoreType.DMA,
        pltpu.SemaphoreType.DMA,
    ],
)
def _sc_scan(l_hbm, t_hbm, pen_hbm, val_hbm, idx_hbm,
             lbuf, tbuf, penv, pv, pi,
             lsem0, lsem1, tsem0, tsem1):
    scid = lax.axis_index("c")
    sidx = lax.axis_index("s")
    wid = sidx * NC + scid
    lane = lax.iota(jnp.int32, LANES)
    off = pl.multiple_of((wid * LAST_TILE_OFF) // (NW - 1) * 128, 128)
    lsems = (lsem0, lsem1)
    tsems = (tsem0, tsem1)

    pltpu.sync_copy(pen_hbm, penv)

    def lsrc(c):
        return l_hbm.at[pl.ds(pl.multiple_of(c * 8, 8), 8), pl.ds(off, STRIPE)]

    def tsrc(c):
        return t_hbm.at[pl.ds(pl.multiple_of(c * 8, 8), 8), pl.ds(off, STRIPE)]

    def start(c, b):
        pltpu.async_copy(lsrc(c), lbuf.at[b], lsems[b])
        pltpu.async_copy(tsrc(c), tbuf.at[b], tsems[b])

    def wait(c, b):
        pltpu.make_async_copy(lsrc(c), lbuf.at[b], lsems[b]).wait()
        pltpu.make_async_copy(tsrc(c), tbuf.at[b], tsems[b]).wait()

    start(0, 0)

    def chunk_body(g, carry):
        vres = jnp.full((LANES,), 0.0, jnp.float32)
        ires = jnp.full((LANES,), 0, jnp.int32)
        for b in range(2):
            c = g * 2 + b
            wait(c, b)

            @pl.when(c + 1 < NCHUNK)
            def _():
                start(c + 1, 1 - b)

            for r8 in range(8):
                row = c * 8 + r8
                pen_s = penv[pl.ds(row * LANES, LANES)]
                rp_s = 1.0 / pen_s

                def body(j, acc, b=b, r8=r8):
                    new = []
                    for u in range(UNROLL):
                        v, i = acc[2 * u], acc[2 * u + 1]
                        col = j * (UNROLL * LANES) + u * LANES
                        l = lbuf[b, r8, pl.ds(col, LANES)]
                        t = tbuf[b, r8, pl.ds(col, LANES)]
                        pm = jnp.minimum(l * rp_s, l * pen_s)
                        p = jnp.where(t > 0, pm, l)
                        gidx = off + col + lane
                        better = jnp.logical_and(p > v, gidx < VOCAB)
                        new.append(jnp.where(better, p, v))
                        new.append(jnp.where(better, gidx, i))
                    return tuple(new)

                acc0 = []
                for _u in range(UNROLL):
                    acc0.append(jnp.full((LANES,), NEG_BIG, jnp.float32))
                    acc0.append(jnp.full((LANES,), 0, jnp.int32))
                acc = lax.fori_loop(0, NSTEP, body, tuple(acc0))

                bv, bi = acc[0], acc[1]
                for u in range(1, UNROLL):
                    v2, i2 = acc[2 * u], acc[2 * u + 1]
                    take = jnp.logical_or(
                        v2 > bv, jnp.logical_and(v2 == bv, i2 < bi))
                    bv = jnp.where(take, v2, bv)
                    bi = jnp.where(take, i2, bi)

                for sh in (1, 2, 4, 8):
                    perm = jnp.bitwise_and(lane + sh, LANES - 1)
                    v2 = jnp.take(bv, perm)
                    i2 = jnp.take(bi, perm)
                    take = jnp.logical_or(
                        v2 > bv, jnp.logical_and(v2 == bv, i2 < bi))
                    bv = jnp.where(take, v2, bv)
                    bi = jnp.where(take, i2, bi)

                rloc = b * 8 + r8
                vres = jnp.where(lane == rloc, bv, vres)
                ires = jnp.where(lane == rloc, bi, ires)
        pv[pl.ds(g * LANES, LANES)] = vres
        pi[pl.ds(g * LANES, LANES)] = ires
        return carry

    lax.fori_loop(0, NCHUNK // 2, chunk_body, 0)

    pltpu.sync_copy(pv, val_hbm.at[pl.ds(wid * BS, BS)])
    pltpu.sync_copy(pi, idx_hbm.at[pl.ds(wid * BS, BS)])


def _merge_body(v_ref, i_ref, o_ref):
    v = v_ref[...]
    i = i_ref[...]
    m = jnp.max(v, axis=0, keepdims=True)
    cand = jnp.where(v == m, i, IDX_BIG)
    o_ref[...] = jnp.min(cand, axis=0, keepdims=True)


def kernel(logits, repetition_penalty, token_count):
    l = logits.reshape(BS, VOCAB)
    t = token_count.reshape(BS, VOCAB)
    pen = jnp.broadcast_to(repetition_penalty.reshape(BS, 1),
                           (BS, LANES)).reshape(BS * LANES)
    vals, idxs = _sc_scan(l, t, pen)
    out = pl.pallas_call(
        _merge_body,
        out_shape=jax.ShapeDtypeStruct((1, BS), jnp.int32),
    )(vals.reshape(NW, BS), idxs.reshape(NW, BS))
    return out.reshape(BS, 1)


# speedup vs baseline: 4.7109x; 1.9838x over previous
import functools
import jax
import jax.numpy as jnp
from jax import lax
from jax.experimental import pallas as pl
from jax.experimental.pallas import tpu as pltpu
from jax.experimental.pallas import tpu_sc as plsc

BS = 128
VOCAB = 100000
LANES = 16
NGRP = BS // LANES
STRIPE = 3200
CROWS = 80
NCHUNK = STRIPE // CROWS
HALF = CROWS // 2

NEG_BIG = -3.0e38
IDX_BIG = 2 ** 30

_info = plsc.get_sparse_core_info()
NC = _info.num_cores
NS = _info.num_subcores
NW = NC * NS
MAX_OFF8 = (VOCAB - STRIPE) // 8

_mesh = plsc.VectorSubcoreMesh(core_axis_name="c", subcore_axis_name="s")


@functools.partial(
    pl.kernel,
    mesh=_mesh,
    out_type=(
        jax.ShapeDtypeStruct((NW * BS,), jnp.float32),
        jax.ShapeDtypeStruct((NW * BS,), jnp.int32),
    ),
    scratch_types=[
        pltpu.VMEM((2, CROWS, BS), jnp.float32),
        pltpu.VMEM((2, CROWS, BS), jnp.int32),
        pltpu.VMEM((BS,), jnp.float32),
        pltpu.VMEM((BS,), jnp.float32),
        pltpu.VMEM((BS,), jnp.int32),
        pltpu.SemaphoreType.DMA,
        pltpu.SemaphoreType.DMA,
        pltpu.SemaphoreType.DMA,
        pltpu.SemaphoreType.DMA,
    ],
)
def _sc_scan(l_hbm, t_hbm, pen_hbm, val_hbm, idx_hbm,
             lbuf, tbuf, penv, pv, pi,
             lsem0, lsem1, tsem0, tsem1):
    scid = lax.axis_index("c")
    sidx = lax.axis_index("s")
    wid = sidx * NC + scid
    off = pl.multiple_of((wid * MAX_OFF8) // (NW - 1) * 8, 8)
    lsems = (lsem0, lsem1)
    tsems = (tsem0, tsem1)

    pltpu.sync_copy(pen_hbm, penv)

    def lsrc(c):
        return l_hbm.at[pl.ds(off + c * CROWS, CROWS), pl.ds(0, BS)]

    def tsrc(c):
        return t_hbm.at[pl.ds(off + c * CROWS, CROWS), pl.ds(0, BS)]

    def start(c, b):
        pltpu.async_copy(lsrc(c), lbuf.at[b], lsems[b])
        pltpu.async_copy(tsrc(c), tbuf.at[b], tsems[b])

    def wait(c, b):
        pltpu.make_async_copy(lsrc(c), lbuf.at[b], lsems[b]).wait()
        pltpu.make_async_copy(tsrc(c), tbuf.at[b], tsems[b]).wait()

    pens = [penv[pl.ds(g * LANES, LANES)] for g in range(NGRP)]
    rps = [1.0 / p for p in pens]

    start(0, 0)

    def chunk_body(gc, carry):
        accs = list(carry)
        for b in range(2):
            c = gc * 2 + b
            wait(c, b)

            @pl.when(c + 1 < NCHUNK)
            def _():
                start(c + 1, 1 - b)

            base = off + c * CROWS
            for g in range(NGRP):
                pen_s = pens[g]
                rp_s = rps[g]

                def vbody(j, a, b=b, g=g, pen_s=pen_s, rp_s=rp_s, base=base):
                    v1, i1, v2, i2 = a
                    col = g * LANES
                    l1 = lbuf[b, j, pl.ds(col, LANES)]
                    t1 = tbuf[b, j, pl.ds(col, LANES)]
                    p1 = jnp.where(t1 > 0,
                                   jnp.minimum(l1 * rp_s, l1 * pen_s), l1)
                    ix1 = jnp.full((LANES,), base + j, jnp.int32)
                    up1 = p1 > v1
                    v1 = jnp.where(up1, p1, v1)
                    i1 = jnp.where(up1, ix1, i1)
                    l2 = lbuf[b, j + HALF, pl.ds(col, LANES)]
                    t2 = tbuf[b, j + HALF, pl.ds(col, LANES)]
                    p2 = jnp.where(t2 > 0,
                                   jnp.minimum(l2 * rp_s, l2 * pen_s), l2)
                    ix2 = jnp.full((LANES,), base + j + HALF, jnp.int32)
                    up2 = p2 > v2
                    v2 = jnp.where(up2, p2, v2)
                    i2 = jnp.where(up2, ix2, i2)
                    return (v1, i1, v2, i2)

                accs[g] = lax.fori_loop(0, HALF, vbody, accs[g])
        return tuple(accs)

    acc0 = []
    for _g in range(NGRP):
        acc0.append((jnp.full((LANES,), NEG_BIG, jnp.float32),
                     jnp.full((LANES,), 0, jnp.int32),
                     jnp.full((LANES,), NEG_BIG, jnp.float32),
                     jnp.full((LANES,), 0, jnp.int32)))
    accs = lax.fori_loop(0, NCHUNK // 2, chunk_body, tuple(acc0))

    for g in range(NGRP):
        v1, i1, v2, i2 = accs[g]
        up = jnp.logical_or(v2 > v1, jnp.logical_and(v2 == v1, i2 < i1))
        pv[pl.ds(g * LANES, LANES)] = jnp.where(up, v2, v1)
        pi[pl.ds(g * LANES, LANES)] = jnp.where(up, i2, i1)

    pltpu.sync_copy(pv, val_hbm.at[pl.ds(wid * BS, BS)])
    pltpu.sync_copy(pi, idx_hbm.at[pl.ds(wid * BS, BS)])


def _merge_body(v_ref, i_ref, o_ref):
    v = v_ref[...]
    i = i_ref[...]
    m = jnp.max(v, axis=0, keepdims=True)
    cand = jnp.where(v == m, i, IDX_BIG)
    o_ref[...] = jnp.min(cand, axis=0, keepdims=True)


def kernel(logits, repetition_penalty, token_count):
    lt = logits.reshape(BS, VOCAB).T
    tt = token_count.T
    pen = repetition_penalty.reshape(BS)
    vals, idxs = _sc_scan(lt, tt, pen)
    out = pl.pallas_call(
        _merge_body,
        out_shape=jax.ShapeDtypeStruct((1, BS), jnp.int32),
    )(vals.reshape(NW, BS), idxs.reshape(NW, BS))
    return out.reshape(BS, 1)
